# baseline (speedup 1.0000x reference)
import functools

import jax
import jax.numpy as jnp
from jax import lax
from jax.experimental import pallas as pl
from jax.experimental.pallas import tpu as pltpu


def _linear_kernel(x_ref, w_ref, b_ref, o_ref):
    o_ref[...] = (
        jnp.dot(x_ref[...], w_ref[...], preferred_element_type=jnp.float32,
                precision=lax.Precision.DEFAULT)
        + b_ref[...]
    ).astype(o_ref.dtype)


@functools.partial(jax.jit, static_argnames=("tm", "tn"))
def _linear_call(x, w_t, b2, tm, tn):
    M, K = x.shape
    _, N = w_t.shape
    grid = (pl.cdiv(M, tm), pl.cdiv(N, tn))
    return pl.pallas_call(
        _linear_kernel,
        out_shape=jax.ShapeDtypeStruct((M, N), jnp.float32),
        grid_spec=pltpu.PrefetchScalarGridSpec(
            num_scalar_prefetch=0,
            grid=grid,
            in_specs=[
                pl.BlockSpec((tm, K), lambda i, j: (i, 0)),
                pl.BlockSpec((K, tn), lambda i, j: (0, j)),
                pl.BlockSpec((1, tn), lambda i, j: (0, j)),
            ],
            out_specs=pl.BlockSpec((tm, tn), lambda i, j: (i, j)),
        ),
        compiler_params=pltpu.CompilerParams(
            dimension_semantics=("parallel", "parallel"),
            vmem_limit_bytes=60 * 1024 * 1024,
        ),
    )(x, w_t, b2)


def kernel(x, w_t, b2):
    return _linear_call(x, w_t, b2, tm=1024, tn=512)
